# initial kernel scaffold (unmeasured)
import jax
import jax.numpy as jnp
from jax import lax
from jax.experimental import pallas as pl
from jax.experimental.pallas import tpu as pltpu

N_DEV = 4
M_BLK = 512


def kernel(x, w_mat):
    m, k = x.shape
    k2, n = w_mat.shape
    nb = m // M_BLK

    def wconv(w_ref, o_ref):
        o_ref[...] = w_ref[...].astype(jnp.bfloat16)

    w_bf = pl.pallas_call(
        wconv,
        out_shape=jax.ShapeDtypeStruct((k2, n), jnp.bfloat16),
        in_specs=[pl.BlockSpec(memory_space=pltpu.VMEM)],
        out_specs=pl.BlockSpec(memory_space=pltpu.VMEM),
    )(w_mat)

    def body(x_ref, w_ref, o_ref, acc_ref, sbuf, rbuf,
             send_sems, recv_sems, credit0, credit1):
        step = pl.program_id(0)
        i = lax.axis_index("i")
        p1 = jnp.bitwise_xor(i, 1)
        p2 = 3 - i

        acc_ref[...] = jnp.dot(
            x_ref[...].astype(jnp.bfloat16), w_ref[...],
            preferred_element_type=jnp.float32,
        )

        for p, partner, credit in ((0, p1, credit0), (1, p2, credit1)):
            sbuf[p, :, :] = acc_ref[...].astype(jnp.bfloat16)

            @pl.when(step > 0)
            def _():
                pl.semaphore_wait(credit, 1)

            rdma = pltpu.make_async_remote_copy(
                src_ref=sbuf.at[p],
                dst_ref=rbuf.at[p],
                send_sem=send_sems.at[p],
                recv_sem=recv_sems.at[p],
                device_id=(partner,),
                device_id_type=pl.DeviceIdType.MESH,
            )
            rdma.start()
            rdma.wait()

            @pl.when(step < nb - 1)
            def _():
                pl.semaphore_signal(
                    credit, inc=1,
                    device_id=(partner,),
                    device_id_type=pl.DeviceIdType.MESH,
                )

            acc_ref[...] += rbuf[p, :, :].astype(jnp.float32)

        o_ref[...] = jax.nn.gelu(acc_ref[...], approximate=True)

    return pl.pallas_call(
        body,
        grid=(nb,),
        out_shape=jax.ShapeDtypeStruct((m, n), jnp.float32),
        in_specs=[
            pl.BlockSpec((M_BLK, k), lambda mm: (mm, 0)),
            pl.BlockSpec((k2, n), lambda mm: (0, 0)),
        ],
        out_specs=pl.BlockSpec((M_BLK, n), lambda mm: (mm, 0)),
        scratch_shapes=[
            pltpu.VMEM((M_BLK, n), jnp.float32),
            pltpu.VMEM((2, M_BLK, n), jnp.bfloat16),
            pltpu.VMEM((2, M_BLK, n), jnp.bfloat16),
            pltpu.SemaphoreType.DMA((2,)),
            pltpu.SemaphoreType.DMA((2,)),
            pltpu.SemaphoreType.REGULAR,
            pltpu.SemaphoreType.REGULAR,
        ],
        compiler_params=pltpu.CompilerParams(collective_id=0),
    )(x, w_bf)


# baseline (device time: 1895225 ns/iter reference)
import jax
import jax.numpy as jnp
from jax import lax
from jax.experimental import pallas as pl
from jax.experimental.pallas import tpu as pltpu

N_DEV = 4
M_BLK = 256


def kernel(x, w_mat):
    m, k = x.shape
    k2, n = w_mat.shape
    nb = m // M_BLK

    def wconv(w_ref, o_ref):
        o_ref[...] = w_ref[...].astype(jnp.bfloat16)

    w_bf = pl.pallas_call(
        wconv,
        out_shape=jax.ShapeDtypeStruct((k2, n), jnp.bfloat16),
        in_specs=[pl.BlockSpec(memory_space=pltpu.VMEM)],
        out_specs=pl.BlockSpec(memory_space=pltpu.VMEM),
    )(w_mat)

    def body(x_ref, w_ref, o_ref, acc_ref, sbuf, rbuf,
             send_sems, recv_sems, credit0, credit1):
        step = pl.program_id(0)
        i = lax.axis_index("i")
        p1 = jnp.bitwise_xor(i, 1)
        p2 = 3 - i

        acc_ref[...] = jnp.dot(
            x_ref[...].astype(jnp.bfloat16), w_ref[...],
            preferred_element_type=jnp.float32,
        )

        for p, partner, credit in ((0, p1, credit0), (1, p2, credit1)):
            sbuf[p, :, :] = acc_ref[...].astype(jnp.bfloat16)

            @pl.when(step > 0)
            def _():
                pl.semaphore_wait(credit, 1)

            rdma = pltpu.make_async_remote_copy(
                src_ref=sbuf.at[p],
                dst_ref=rbuf.at[p],
                send_sem=send_sems.at[p],
                recv_sem=recv_sems.at[p],
                device_id=(partner,),
                device_id_type=pl.DeviceIdType.MESH,
            )
            rdma.start()
            rdma.wait()

            @pl.when(step < nb - 1)
            def _():
                pl.semaphore_signal(
                    credit, inc=1,
                    device_id=(partner,),
                    device_id_type=pl.DeviceIdType.MESH,
                )

            acc_ref[...] += rbuf[p, :, :].astype(jnp.float32)

        o_ref[...] = jax.nn.gelu(acc_ref[...], approximate=True)

    return pl.pallas_call(
        body,
        grid=(nb,),
        out_shape=jax.ShapeDtypeStruct((m, n), jnp.float32),
        in_specs=[
            pl.BlockSpec((M_BLK, k), lambda mm: (mm, 0)),
            pl.BlockSpec((k2, n), lambda mm: (0, 0)),
        ],
        out_specs=pl.BlockSpec((M_BLK, n), lambda mm: (mm, 0)),
        scratch_shapes=[
            pltpu.VMEM((M_BLK, n), jnp.float32),
            pltpu.VMEM((2, M_BLK, n), jnp.bfloat16),
            pltpu.VMEM((2, M_BLK, n), jnp.bfloat16),
            pltpu.SemaphoreType.DMA((2,)),
            pltpu.SemaphoreType.DMA((2,)),
            pltpu.SemaphoreType.REGULAR,
            pltpu.SemaphoreType.REGULAR,
        ],
        compiler_params=pltpu.CompilerParams(
            vmem_limit_bytes=100 * 1024 * 1024,
        ),
    )(x, w_bf)


# device time: 1145234 ns/iter; 1.6549x vs baseline; 1.6549x over previous
import jax
import jax.numpy as jnp
from jax import lax
from jax.experimental import pallas as pl
from jax.experimental.pallas import tpu as pltpu

N_DEV = 4
M_BLK = 512
NHALF = 2048


def _convert_bf16(a, blk_rows):
    rows, cols = a.shape

    def conv(a_ref, o_ref):
        o_ref[...] = a_ref[...].astype(jnp.bfloat16)

    return pl.pallas_call(
        conv,
        grid=(rows // blk_rows,),
        out_shape=jax.ShapeDtypeStruct((rows, cols), jnp.bfloat16),
        in_specs=[pl.BlockSpec((blk_rows, cols), lambda r: (r, 0))],
        out_specs=pl.BlockSpec((blk_rows, cols), lambda r: (r, 0)),
    )(a)


def kernel(x, w_mat):
    m, k = x.shape
    k2, n = w_mat.shape
    nb = m // M_BLK

    x_bf = _convert_bf16(x, 512)
    w_bf = _convert_bf16(w_mat, 256)

    def body(x_ref, w_ref, o_ref, sbufA, sbufB, rbuf,
             send_sems, recv_sems, credit1, credit2):
        step = pl.program_id(0)
        i = lax.axis_index("i")
        p1 = jnp.bitwise_xor(i, 1)
        p2 = 3 - i

        sbufA[...] = jnp.dot(
            x_ref[...], w_ref[:, :NHALF], preferred_element_type=jnp.float32
        ).astype(jnp.bfloat16)
        sbufB[...] = jnp.dot(
            x_ref[...], w_ref[:, NHALF:], preferred_element_type=jnp.float32
        ).astype(jnp.bfloat16)

        @pl.when(step > 0)
        def _():
            pl.semaphore_wait(credit1, 1)
            pl.semaphore_wait(credit2, 1)

        def exchange(slot, sbuf, partner):
            return pltpu.make_async_remote_copy(
                src_ref=sbuf.at[...],
                dst_ref=rbuf.at[slot],
                send_sem=send_sems.at[slot],
                recv_sem=recv_sems.at[slot],
                device_id=(partner,),
                device_id_type=pl.DeviceIdType.MESH,
            )

        rdma_a0 = exchange(0, sbufA, p1)
        rdma_b0 = exchange(1, sbufB, p2)
        rdma_a0.start()
        rdma_b0.start()
        rdma_a0.wait()
        rdma_b0.wait()

        sbufA[...] = (
            sbufA[...].astype(jnp.float32) + rbuf[0].astype(jnp.float32)
        ).astype(jnp.bfloat16)
        sbufB[...] = (
            sbufB[...].astype(jnp.float32) + rbuf[1].astype(jnp.float32)
        ).astype(jnp.bfloat16)

        rdma_a1 = exchange(2, sbufA, p2)
        rdma_b1 = exchange(3, sbufB, p1)
        rdma_a1.start()
        rdma_b1.start()
        rdma_a1.wait()
        rdma_b1.wait()

        yA = sbufA[...].astype(jnp.float32) + rbuf[2].astype(jnp.float32)
        yB = sbufB[...].astype(jnp.float32) + rbuf[3].astype(jnp.float32)
        o_ref[:, :NHALF] = jax.nn.gelu(yA, approximate=True)
        o_ref[:, NHALF:] = jax.nn.gelu(yB, approximate=True)

        @pl.when(step < nb - 1)
        def _():
            pl.semaphore_signal(
                credit1, inc=1, device_id=(p1,),
                device_id_type=pl.DeviceIdType.MESH,
            )
            pl.semaphore_signal(
                credit2, inc=1, device_id=(p2,),
                device_id_type=pl.DeviceIdType.MESH,
            )

    return pl.pallas_call(
        body,
        grid=(nb,),
        out_shape=jax.ShapeDtypeStruct((m, n), jnp.float32),
        in_specs=[
            pl.BlockSpec((M_BLK, k), lambda mm: (mm, 0)),
            pl.BlockSpec((k2, n), lambda mm: (0, 0)),
        ],
        out_specs=pl.BlockSpec((M_BLK, n), lambda mm: (mm, 0)),
        scratch_shapes=[
            pltpu.VMEM((M_BLK, NHALF), jnp.bfloat16),
            pltpu.VMEM((M_BLK, NHALF), jnp.bfloat16),
            pltpu.VMEM((4, M_BLK, NHALF), jnp.bfloat16),
            pltpu.SemaphoreType.DMA((4,)),
            pltpu.SemaphoreType.DMA((4,)),
            pltpu.SemaphoreType.REGULAR,
            pltpu.SemaphoreType.REGULAR,
        ],
        compiler_params=pltpu.CompilerParams(
            vmem_limit_bytes=100 * 1024 * 1024,
        ),
    )(x_bf, w_bf)


# device time: 1017816 ns/iter; 1.8621x vs baseline; 1.1252x over previous
import jax
import jax.numpy as jnp
from jax import lax
from jax.experimental import pallas as pl
from jax.experimental.pallas import tpu as pltpu

N_DEV = 4
M_BLK = 512
NHALF = 2048
F32 = jnp.float32
BF16 = jnp.bfloat16


def _convert_bf16(a, blk_rows):
    rows, cols = a.shape

    def conv(a_ref, o_ref):
        o_ref[...] = a_ref[...].astype(BF16)

    return pl.pallas_call(
        conv,
        grid=(rows // blk_rows,),
        out_shape=jax.ShapeDtypeStruct((rows, cols), BF16),
        in_specs=[pl.BlockSpec((blk_rows, cols), lambda r: (r, 0))],
        out_specs=pl.BlockSpec((blk_rows, cols), lambda r: (r, 0)),
    )(a)


def kernel(x, w_mat):
    m, k = x.shape
    k2, n = w_mat.shape
    nb = m // M_BLK

    x_bf = _convert_bf16(x, 512)
    w_bf = _convert_bf16(w_mat, 256)

    def body(x_ref, w_ref, o_ref,
             pcompA, pcompB, sbuf0A, sbuf0B, sbuf1A, sbuf1B, rbuf,
             send_sems, recv_sems, c1, c2):
        s = pl.program_id(0)
        i = lax.axis_index("i")
        p1 = jnp.bitwise_xor(i, 1)
        p2 = 3 - i

        def rdma(slot, sbuf, partner):
            return pltpu.make_async_remote_copy(
                src_ref=sbuf.at[...],
                dst_ref=rbuf.at[slot],
                send_sem=send_sems.at[slot],
                recv_sem=recv_sems.at[slot],
                device_id=(partner,),
                device_id_type=pl.DeviceIdType.MESH,
            )

        ph0A = rdma(0, sbuf0A, p1)
        ph0B = rdma(1, sbuf0B, p2)
        ph1A = rdma(2, sbuf1A, p2)
        ph1B = rdma(3, sbuf1B, p1)

        @pl.when((s >= 2) & (s <= nb))
        def _():
            pl.semaphore_wait(c1, 1)
            pl.semaphore_wait(c2, 1)

        @pl.when((s >= 1) & (s <= nb))
        def _():
            sbuf0A[...] = pcompA[...]
            sbuf0B[...] = pcompB[...]
            ph0A.start()
            ph0B.start()

        @pl.when(s <= nb - 1)
        def _():
            pcompA[...] = jnp.dot(
                x_ref[...], w_ref[:, :NHALF], preferred_element_type=F32
            ).astype(BF16)
            pcompB[...] = jnp.dot(
                x_ref[...], w_ref[:, NHALF:], preferred_element_type=F32
            ).astype(BF16)

        @pl.when(s >= 1)
        def _():
            ph0A.wait()
            ph0B.wait()
            sbuf1A[...] = (
                sbuf0A[...].astype(F32) + rbuf[0].astype(F32)
            ).astype(BF16)
            sbuf1B[...] = (
                sbuf0B[...].astype(F32) + rbuf[1].astype(F32)
            ).astype(BF16)

            ph1A.start()
            ph1B.start()
            ph1A.wait()
            ph1B.wait()

            yA = sbuf1A[...].astype(F32) + rbuf[2].astype(F32)
            yB = sbuf1B[...].astype(F32) + rbuf[3].astype(F32)
            o_ref[:, :NHALF] = jax.nn.gelu(yA, approximate=True)
            o_ref[:, NHALF:] = jax.nn.gelu(yB, approximate=True)

        @pl.when((s >= 1) & (s <= nb - 1))
        def _():
            pl.semaphore_signal(
                c1, inc=1, device_id=(p1,),
                device_id_type=pl.DeviceIdType.MESH,
            )
            pl.semaphore_signal(
                c2, inc=1, device_id=(p2,),
                device_id_type=pl.DeviceIdType.MESH,
            )

    return pl.pallas_call(
        body,
        grid=(nb + 1,),
        out_shape=jax.ShapeDtypeStruct((m, n), F32),
        in_specs=[
            pl.BlockSpec((M_BLK, k), lambda s: (jnp.minimum(s, nb - 1), 0)),
            pl.BlockSpec((k2, n), lambda s: (0, 0)),
        ],
        out_specs=pl.BlockSpec(
            (M_BLK, n), lambda s: (jnp.maximum(s - 1, 0), 0)
        ),
        scratch_shapes=[
            pltpu.VMEM((M_BLK, NHALF), BF16),
            pltpu.VMEM((M_BLK, NHALF), BF16),
            pltpu.VMEM((M_BLK, NHALF), BF16),
            pltpu.VMEM((M_BLK, NHALF), BF16),
            pltpu.VMEM((M_BLK, NHALF), BF16),
            pltpu.VMEM((M_BLK, NHALF), BF16),
            pltpu.VMEM((4, M_BLK, NHALF), BF16),
            pltpu.SemaphoreType.DMA((4,)),
            pltpu.SemaphoreType.DMA((4,)),
            pltpu.SemaphoreType.REGULAR,
            pltpu.SemaphoreType.REGULAR,
        ],
        compiler_params=pltpu.CompilerParams(
            vmem_limit_bytes=100 * 1024 * 1024,
        ),
    )(x_bf, w_bf)


# device time: 1015431 ns/iter; 1.8664x vs baseline; 1.0023x over previous
import jax
import jax.numpy as jnp
from jax import lax
from jax.experimental import pallas as pl
from jax.experimental.pallas import tpu as pltpu

N_DEV = 4
M_BLK = 512
NHALF = 2048
F32 = jnp.float32
BF16 = jnp.bfloat16


def _convert_bf16(a, blk_rows):
    rows, cols = a.shape

    def conv(a_ref, o_ref):
        o_ref[...] = a_ref[...].astype(BF16)

    return pl.pallas_call(
        conv,
        grid=(rows // blk_rows,),
        out_shape=jax.ShapeDtypeStruct((rows, cols), BF16),
        in_specs=[pl.BlockSpec((blk_rows, cols), lambda r: (r, 0))],
        out_specs=pl.BlockSpec((blk_rows, cols), lambda r: (r, 0)),
    )(a)


def kernel(x, w_mat):
    m, k = x.shape
    k2, n = w_mat.shape
    nb = m // M_BLK

    x_bf = _convert_bf16(x, 512)
    w_bf = _convert_bf16(w_mat, 256)

    def body(x_ref, w_ref, o_ref,
             pcompA, pcompB, sbuf0A, sbuf0B, sbuf1A, sbuf1B, rbuf,
             send_sems, recv_sems, c1, c2):
        s = pl.program_id(0)
        i = lax.axis_index("i")
        p1 = jnp.bitwise_xor(i, 1)
        p2 = 3 - i

        def rdma(slot, sbuf, partner):
            return pltpu.make_async_remote_copy(
                src_ref=sbuf.at[...],
                dst_ref=rbuf.at[slot],
                send_sem=send_sems.at[slot],
                recv_sem=recv_sems.at[slot],
                device_id=(partner,),
                device_id_type=pl.DeviceIdType.MESH,
            )

        ph0A = rdma(0, sbuf0A, p1)
        ph0B = rdma(1, sbuf0B, p2)
        ph1A = rdma(2, sbuf1A, p2)
        ph1B = rdma(3, sbuf1B, p1)

        @pl.when(s == 0)
        def _():
            bsem = pltpu.get_barrier_semaphore()
            pl.semaphore_signal(
                bsem, inc=1, device_id=(p1,),
                device_id_type=pl.DeviceIdType.MESH,
            )
            pl.semaphore_signal(
                bsem, inc=1, device_id=(p2,),
                device_id_type=pl.DeviceIdType.MESH,
            )
            pl.semaphore_wait(bsem, 2)

        @pl.when((s >= 2) & (s <= nb))
        def _():
            pl.semaphore_wait(c1, 1)
            pl.semaphore_wait(c2, 1)

        @pl.when((s >= 1) & (s <= nb))
        def _():
            sbuf0A[...] = pcompA[...]
            sbuf0B[...] = pcompB[...]
            ph0A.start()
            ph0B.start()

        @pl.when(s <= nb - 1)
        def _():
            pcompA[...] = jnp.dot(
                x_ref[...], w_ref[:, :NHALF], preferred_element_type=F32
            ).astype(BF16)
            pcompB[...] = jnp.dot(
                x_ref[...], w_ref[:, NHALF:], preferred_element_type=F32
            ).astype(BF16)

        @pl.when(s >= 1)
        def _():
            ph0A.wait()
            ph0B.wait()
            sbuf1A[...] = (
                sbuf0A[...].astype(F32) + rbuf[0].astype(F32)
            ).astype(BF16)
            sbuf1B[...] = (
                sbuf0B[...].astype(F32) + rbuf[1].astype(F32)
            ).astype(BF16)

            ph1A.start()
            ph1B.start()
            ph1A.wait()
            ph1B.wait()

            yA = sbuf1A[...].astype(F32) + rbuf[2].astype(F32)
            yB = sbuf1B[...].astype(F32) + rbuf[3].astype(F32)
            o_ref[:, :NHALF] = jax.nn.gelu(yA, approximate=True)
            o_ref[:, NHALF:] = jax.nn.gelu(yB, approximate=True)

        @pl.when((s >= 1) & (s <= nb - 1))
        def _():
            pl.semaphore_signal(
                c1, inc=1, device_id=(p1,),
                device_id_type=pl.DeviceIdType.MESH,
            )
            pl.semaphore_signal(
                c2, inc=1, device_id=(p2,),
                device_id_type=pl.DeviceIdType.MESH,
            )

    return pl.pallas_call(
        body,
        grid=(nb + 1,),
        out_shape=jax.ShapeDtypeStruct((m, n), F32),
        in_specs=[
            pl.BlockSpec((M_BLK, k), lambda s: (jnp.minimum(s, nb - 1), 0)),
            pl.BlockSpec((k2, n), lambda s: (0, 0)),
        ],
        out_specs=pl.BlockSpec(
            (M_BLK, n), lambda s: (jnp.maximum(s - 1, 0), 0)
        ),
        scratch_shapes=[
            pltpu.VMEM((M_BLK, NHALF), BF16),
            pltpu.VMEM((M_BLK, NHALF), BF16),
            pltpu.VMEM((M_BLK, NHALF), BF16),
            pltpu.VMEM((M_BLK, NHALF), BF16),
            pltpu.VMEM((M_BLK, NHALF), BF16),
            pltpu.VMEM((M_BLK, NHALF), BF16),
            pltpu.VMEM((4, M_BLK, NHALF), BF16),
            pltpu.SemaphoreType.DMA((4,)),
            pltpu.SemaphoreType.DMA((4,)),
            pltpu.SemaphoreType.REGULAR,
            pltpu.SemaphoreType.REGULAR,
        ],
        compiler_params=pltpu.CompilerParams(
            vmem_limit_bytes=100 * 1024 * 1024,
            collective_id=0,
        ),
    )(x_bf, w_bf)


# device time: 947405 ns/iter; 2.0004x vs baseline; 1.0718x over previous
import jax
import jax.numpy as jnp
from jax import lax
from jax.experimental import pallas as pl
from jax.experimental.pallas import tpu as pltpu

N_DEV = 4
M_BLK = 512
SUB = 256
NHALF = 2048
F32 = jnp.float32
BF16 = jnp.bfloat16


def _convert_bf16(a, blk_rows):
    rows, cols = a.shape

    def conv(a_ref, o_ref):
        o_ref[...] = a_ref[...].astype(BF16)

    return pl.pallas_call(
        conv,
        grid=(rows // blk_rows,),
        out_shape=jax.ShapeDtypeStruct((rows, cols), BF16),
        in_specs=[pl.BlockSpec((blk_rows, cols), lambda r: (r, 0))],
        out_specs=pl.BlockSpec((blk_rows, cols), lambda r: (r, 0)),
    )(a)


def kernel(x, w_mat):
    m, k = x.shape
    k2, n = w_mat.shape
    nb = m // M_BLK

    x_bf = _convert_bf16(x, 512)
    w_bf = _convert_bf16(w_mat, 256)

    def body(x_ref, w_ref, o_ref,
             pcompA, pcompB, sbuf0, sbuf1, rbuf,
             send_sems, recv_sems, c1, c2):
        s = pl.program_id(0)
        i = lax.axis_index("i")
        p1 = jnp.bitwise_xor(i, 1)
        p2 = 3 - i

        def partner(phase, half):
            if (phase + half) % 2 == 0:
                return p1
            return p2

        def rdma(phase, sub, half, sbuf):
            return pltpu.make_async_remote_copy(
                src_ref=sbuf.at[sub, half],
                dst_ref=rbuf.at[phase, sub, half],
                send_sem=send_sems.at[phase, sub, half],
                recv_sem=recv_sems.at[phase, sub, half],
                device_id=(partner(phase, half),),
                device_id_type=pl.DeviceIdType.MESH,
            )

        @pl.when(s == 0)
        def _():
            bsem = pltpu.get_barrier_semaphore()
            pl.semaphore_signal(
                bsem, inc=1, device_id=(p1,),
                device_id_type=pl.DeviceIdType.MESH,
            )
            pl.semaphore_signal(
                bsem, inc=1, device_id=(p2,),
                device_id_type=pl.DeviceIdType.MESH,
            )
            pl.semaphore_wait(bsem, 2)

        @pl.when((s >= 2) & (s <= nb))
        def _():
            pl.semaphore_wait(c1, 1)
            pl.semaphore_wait(c2, 1)

        @pl.when((s >= 1) & (s <= nb))
        def _():
            for sub in (0, 1):
                r0 = sub * SUB
                sbuf0[sub, 0] = pcompA[r0:r0 + SUB, :]
                sbuf0[sub, 1] = pcompB[r0:r0 + SUB, :]
            for sub in (0, 1):
                rdma(0, sub, 0, sbuf0).start()
                rdma(0, sub, 1, sbuf0).start()

        @pl.when(s <= nb - 1)
        def _():
            pcompA[...] = jnp.dot(
                x_ref[...], w_ref[:, :NHALF], preferred_element_type=F32
            ).astype(BF16)
            pcompB[...] = jnp.dot(
                x_ref[...], w_ref[:, NHALF:], preferred_element_type=F32
            ).astype(BF16)

        @pl.when(s >= 1)
        def _():
            for sub in (0, 1):
                for half in (0, 1):
                    rdma(0, sub, half, sbuf0).wait()
                for half in (0, 1):
                    sbuf1[sub, half] = (
                        sbuf0[sub, half].astype(F32)
                        + rbuf[0, sub, half].astype(F32)
                    ).astype(BF16)
                for half in (0, 1):
                    rdma(1, sub, half, sbuf1).start()

            for sub in (0, 1):
                r0 = sub * SUB
                for half in (0, 1):
                    rdma(1, sub, half, sbuf1).wait()
                yA = (
                    sbuf1[sub, 0].astype(F32) + rbuf[1, sub, 0].astype(F32)
                )
                yB = (
                    sbuf1[sub, 1].astype(F32) + rbuf[1, sub, 1].astype(F32)
                )
                o_ref[r0:r0 + SUB, :NHALF] = jax.nn.gelu(
                    yA, approximate=True
                )
                o_ref[r0:r0 + SUB, NHALF:] = jax.nn.gelu(
                    yB, approximate=True
                )

        @pl.when((s >= 1) & (s <= nb - 1))
        def _():
            pl.semaphore_signal(
                c1, inc=1, device_id=(p1,),
                device_id_type=pl.DeviceIdType.MESH,
            )
            pl.semaphore_signal(
                c2, inc=1, device_id=(p2,),
                device_id_type=pl.DeviceIdType.MESH,
            )

    return pl.pallas_call(
        body,
        grid=(nb + 1,),
        out_shape=jax.ShapeDtypeStruct((m, n), F32),
        in_specs=[
            pl.BlockSpec((M_BLK, k), lambda s: (jnp.minimum(s, nb - 1), 0)),
            pl.BlockSpec((k2, n), lambda s: (0, 0)),
        ],
        out_specs=pl.BlockSpec(
            (M_BLK, n), lambda s: (jnp.maximum(s - 1, 0), 0)
        ),
        scratch_shapes=[
            pltpu.VMEM((M_BLK, NHALF), BF16),
            pltpu.VMEM((M_BLK, NHALF), BF16),
            pltpu.VMEM((2, 2, SUB, NHALF), BF16),
            pltpu.VMEM((2, 2, SUB, NHALF), BF16),
            pltpu.VMEM((2, 2, 2, SUB, NHALF), BF16),
            pltpu.SemaphoreType.DMA((2, 2, 2)),
            pltpu.SemaphoreType.DMA((2, 2, 2)),
            pltpu.SemaphoreType.REGULAR,
            pltpu.SemaphoreType.REGULAR,
        ],
        compiler_params=pltpu.CompilerParams(
            vmem_limit_bytes=100 * 1024 * 1024,
            collective_id=0,
        ),
    )(x_bf, w_bf)


# device time: 934607 ns/iter; 2.0278x vs baseline; 1.0137x over previous
import jax
import jax.numpy as jnp
from jax import lax
from jax.experimental import pallas as pl
from jax.experimental.pallas import tpu as pltpu

N_DEV = 4
M_BLK = 512
SUB = 128
NHALF = 2048
F32 = jnp.float32
BF16 = jnp.bfloat16


def _convert_bf16(a, blk_rows):
    rows, cols = a.shape

    def conv(a_ref, o_ref):
        o_ref[...] = a_ref[...].astype(BF16)

    return pl.pallas_call(
        conv,
        grid=(rows // blk_rows,),
        out_shape=jax.ShapeDtypeStruct((rows, cols), BF16),
        in_specs=[pl.BlockSpec((blk_rows, cols), lambda r: (r, 0))],
        out_specs=pl.BlockSpec((blk_rows, cols), lambda r: (r, 0)),
    )(a)


def kernel(x, w_mat):
    m, k = x.shape
    k2, n = w_mat.shape
    nb = m // M_BLK

    x_bf = _convert_bf16(x, 512)
    w_bf = _convert_bf16(w_mat, 256)

    def body(x_ref, w_ref, o_ref,
             pcompA, pcompB, sbuf0, sbuf1, rbuf,
             send_sems, recv_sems, c1, c2):
        s = pl.program_id(0)
        i = lax.axis_index("i")
        p1 = jnp.bitwise_xor(i, 1)
        p2 = 3 - i

        def partner(phase, half):
            if (phase + half) % 2 == 0:
                return p1
            return p2

        def rdma(phase, sub, half, sbuf):
            return pltpu.make_async_remote_copy(
                src_ref=sbuf.at[sub, half],
                dst_ref=rbuf.at[phase, sub, half],
                send_sem=send_sems.at[phase, sub, half],
                recv_sem=recv_sems.at[phase, sub, half],
                device_id=(partner(phase, half),),
                device_id_type=pl.DeviceIdType.MESH,
            )

        @pl.when(s == 0)
        def _():
            bsem = pltpu.get_barrier_semaphore()
            pl.semaphore_signal(
                bsem, inc=1, device_id=(p1,),
                device_id_type=pl.DeviceIdType.MESH,
            )
            pl.semaphore_signal(
                bsem, inc=1, device_id=(p2,),
                device_id_type=pl.DeviceIdType.MESH,
            )
            pl.semaphore_wait(bsem, 2)

        @pl.when((s >= 2) & (s <= nb))
        def _():
            pl.semaphore_wait(c1, 1)
            pl.semaphore_wait(c2, 1)

        @pl.when((s >= 1) & (s <= nb))
        def _():
            for sub in range(4):
                r0 = sub * SUB
                sbuf0[sub, 0] = pcompA[r0:r0 + SUB, :]
                sbuf0[sub, 1] = pcompB[r0:r0 + SUB, :]
            for sub in range(4):
                rdma(0, sub, 0, sbuf0).start()
                rdma(0, sub, 1, sbuf0).start()

        @pl.when(s <= nb - 1)
        def _():
            pcompA[...] = jnp.dot(
                x_ref[...], w_ref[:, :NHALF], preferred_element_type=F32
            ).astype(BF16)
            pcompB[...] = jnp.dot(
                x_ref[...], w_ref[:, NHALF:], preferred_element_type=F32
            ).astype(BF16)

        @pl.when(s >= 1)
        def _():
            for sub in range(4):
                for half in (0, 1):
                    rdma(0, sub, half, sbuf0).wait()
                for half in (0, 1):
                    sbuf1[sub, half] = (
                        sbuf0[sub, half].astype(F32)
                        + rbuf[0, sub, half].astype(F32)
                    ).astype(BF16)
                for half in (0, 1):
                    rdma(1, sub, half, sbuf1).start()

            for sub in range(4):
                r0 = sub * SUB
                for half in (0, 1):
                    rdma(1, sub, half, sbuf1).wait()
                yA = (
                    sbuf1[sub, 0].astype(F32) + rbuf[1, sub, 0].astype(F32)
                )
                yB = (
                    sbuf1[sub, 1].astype(F32) + rbuf[1, sub, 1].astype(F32)
                )
                o_ref[r0:r0 + SUB, :NHALF] = jax.nn.gelu(
                    yA, approximate=True
                )
                o_ref[r0:r0 + SUB, NHALF:] = jax.nn.gelu(
                    yB, approximate=True
                )

        @pl.when((s >= 1) & (s <= nb - 1))
        def _():
            pl.semaphore_signal(
                c1, inc=1, device_id=(p1,),
                device_id_type=pl.DeviceIdType.MESH,
            )
            pl.semaphore_signal(
                c2, inc=1, device_id=(p2,),
                device_id_type=pl.DeviceIdType.MESH,
            )

    return pl.pallas_call(
        body,
        grid=(nb + 1,),
        out_shape=jax.ShapeDtypeStruct((m, n), F32),
        in_specs=[
            pl.BlockSpec((M_BLK, k), lambda s: (jnp.minimum(s, nb - 1), 0)),
            pl.BlockSpec((k2, n), lambda s: (0, 0)),
        ],
        out_specs=pl.BlockSpec(
            (M_BLK, n), lambda s: (jnp.maximum(s - 1, 0), 0)
        ),
        scratch_shapes=[
            pltpu.VMEM((M_BLK, NHALF), BF16),
            pltpu.VMEM((M_BLK, NHALF), BF16),
            pltpu.VMEM((4, 2, SUB, NHALF), BF16),
            pltpu.VMEM((4, 2, SUB, NHALF), BF16),
            pltpu.VMEM((2, 4, 2, SUB, NHALF), BF16),
            pltpu.SemaphoreType.DMA((2, 4, 2)),
            pltpu.SemaphoreType.DMA((2, 4, 2)),
            pltpu.SemaphoreType.REGULAR,
            pltpu.SemaphoreType.REGULAR,
        ],
        compiler_params=pltpu.CompilerParams(
            vmem_limit_bytes=100 * 1024 * 1024,
            collective_id=0,
        ),
    )(x_bf, w_bf)


# device time: 923150 ns/iter; 2.0530x vs baseline; 1.0124x over previous
import jax
import jax.numpy as jnp
from jax import lax
from jax.experimental import pallas as pl
from jax.experimental.pallas import tpu as pltpu

N_DEV = 4
M_BLK = 512
SUB = 128
NHALF = 2048
F32 = jnp.float32
BF16 = jnp.bfloat16


def _convert_bf16(a, blk_rows):
    rows, cols = a.shape

    def conv(a_ref, o_ref):
        o_ref[...] = a_ref[...].astype(BF16)

    return pl.pallas_call(
        conv,
        grid=(rows // blk_rows,),
        out_shape=jax.ShapeDtypeStruct((rows, cols), BF16),
        in_specs=[pl.BlockSpec((blk_rows, cols), lambda r: (r, 0))],
        out_specs=pl.BlockSpec((blk_rows, cols), lambda r: (r, 0)),
    )(a)


def kernel(x, w_mat):
    m, k = x.shape
    k2, n = w_mat.shape
    nb = m // M_BLK

    x_bf = _convert_bf16(x, 512)
    w_bf = _convert_bf16(w_mat, 256)

    def body(x_ref, w_ref, o_ref,
             pcompA, pcompB, sbuf0, sbuf1, rbuf,
             send_sems, recv_sems, c1, c2):
        s = pl.program_id(0)
        i = lax.axis_index("i")
        p1 = jnp.bitwise_xor(i, 1)
        p2 = 3 - i

        def partner(phase, half):
            if (phase + half) % 2 == 0:
                return p1
            return p2

        def rdma(phase, sub, half, sbuf):
            return pltpu.make_async_remote_copy(
                src_ref=sbuf.at[sub, half],
                dst_ref=rbuf.at[phase, sub, half],
                send_sem=send_sems.at[phase, sub, half],
                recv_sem=recv_sems.at[phase, sub, half],
                device_id=(partner(phase, half),),
                device_id_type=pl.DeviceIdType.MESH,
            )

        @pl.when(s == 0)
        def _():
            bsem = pltpu.get_barrier_semaphore()
            pl.semaphore_signal(
                bsem, inc=1, device_id=(p1,),
                device_id_type=pl.DeviceIdType.MESH,
            )
            pl.semaphore_signal(
                bsem, inc=1, device_id=(p2,),
                device_id_type=pl.DeviceIdType.MESH,
            )
            pl.semaphore_wait(bsem, 2)

        @pl.when((s >= 2) & (s <= nb))
        def _():
            pl.semaphore_wait(c1, 1)
            pl.semaphore_wait(c2, 1)

        @pl.when((s >= 1) & (s <= nb))
        def _():
            for sub in range(4):
                r0 = sub * SUB
                sbuf0[sub, 0] = pcompA[r0:r0 + SUB, :]
                sbuf0[sub, 1] = pcompB[r0:r0 + SUB, :]
            for sub in range(4):
                rdma(0, sub, 0, sbuf0).start()
                rdma(0, sub, 1, sbuf0).start()

        @pl.when(s >= 1)
        def _():
            for sub in range(4):
                for half in (0, 1):
                    rdma(0, sub, half, sbuf0).wait()
                for half in (0, 1):
                    sbuf1[sub, half] = (
                        sbuf0[sub, half].astype(F32)
                        + rbuf[0, sub, half].astype(F32)
                    ).astype(BF16)
                for half in (0, 1):
                    rdma(1, sub, half, sbuf1).start()

        @pl.when((s >= 1) & (s <= nb - 1))
        def _():
            pl.semaphore_signal(
                c1, inc=1, device_id=(p1,),
                device_id_type=pl.DeviceIdType.MESH,
            )
            pl.semaphore_signal(
                c2, inc=1, device_id=(p2,),
                device_id_type=pl.DeviceIdType.MESH,
            )

        @pl.when(s <= nb - 1)
        def _():
            pcompA[...] = jnp.dot(
                x_ref[...], w_ref[:, :NHALF], preferred_element_type=F32
            ).astype(BF16)
            pcompB[...] = jnp.dot(
                x_ref[...], w_ref[:, NHALF:], preferred_element_type=F32
            ).astype(BF16)

        @pl.when(s >= 1)
        def _():
            for sub in range(4):
                r0 = sub * SUB
                for half in (0, 1):
                    rdma(1, sub, half, sbuf1).wait()
                yA = (
                    sbuf1[sub, 0].astype(F32) + rbuf[1, sub, 0].astype(F32)
                )
                yB = (
                    sbuf1[sub, 1].astype(F32) + rbuf[1, sub, 1].astype(F32)
                )
                o_ref[r0:r0 + SUB, :NHALF] = jax.nn.gelu(
                    yA, approximate=True
                )
                o_ref[r0:r0 + SUB, NHALF:] = jax.nn.gelu(
                    yB, approximate=True
                )

    return pl.pallas_call(
        body,
        grid=(nb + 1,),
        out_shape=jax.ShapeDtypeStruct((m, n), F32),
        in_specs=[
            pl.BlockSpec((M_BLK, k), lambda s: (jnp.minimum(s, nb - 1), 0)),
            pl.BlockSpec((k2, n), lambda s: (0, 0)),
        ],
        out_specs=pl.BlockSpec(
            (M_BLK, n), lambda s: (jnp.maximum(s - 1, 0), 0)
        ),
        scratch_shapes=[
            pltpu.VMEM((M_BLK, NHALF), BF16),
            pltpu.VMEM((M_BLK, NHALF), BF16),
            pltpu.VMEM((4, 2, SUB, NHALF), BF16),
            pltpu.VMEM((4, 2, SUB, NHALF), BF16),
            pltpu.VMEM((2, 4, 2, SUB, NHALF), BF16),
            pltpu.SemaphoreType.DMA((2, 4, 2)),
            pltpu.SemaphoreType.DMA((2, 4, 2)),
            pltpu.SemaphoreType.REGULAR,
            pltpu.SemaphoreType.REGULAR,
        ],
        compiler_params=pltpu.CompilerParams(
            vmem_limit_bytes=100 * 1024 * 1024,
            collective_id=0,
        ),
    )(x_bf, w_bf)


# device time: 918786 ns/iter; 2.0627x vs baseline; 1.0047x over previous
import jax
import jax.numpy as jnp
from jax import lax
from jax.experimental import pallas as pl
from jax.experimental.pallas import tpu as pltpu

N_DEV = 4
M_BLK = 512
SUB = 128
NHALF = 2048
F32 = jnp.float32
BF16 = jnp.bfloat16


def _convert_bf16(a, blk_rows):
    rows, cols = a.shape

    def conv(a_ref, o_ref):
        o_ref[...] = a_ref[...].astype(BF16)

    return pl.pallas_call(
        conv,
        grid=(rows // blk_rows,),
        out_shape=jax.ShapeDtypeStruct((rows, cols), BF16),
        in_specs=[pl.BlockSpec((blk_rows, cols), lambda r: (r, 0))],
        out_specs=pl.BlockSpec((blk_rows, cols), lambda r: (r, 0)),
    )(a)


def kernel(x, w_mat):
    m, k = x.shape
    k2, n = w_mat.shape
    nb = m // M_BLK

    x_bf = _convert_bf16(x, 512)
    w_bf = _convert_bf16(w_mat, 256)

    def body(x_ref, w_ref, o_ref,
             sbuf0, sbuf1, rbuf,
             send_sems, recv_sems, c1, c2):
        s = pl.program_id(0)
        i = lax.axis_index("i")
        p1 = jnp.bitwise_xor(i, 1)
        p2 = 3 - i

        def partner(phase, half):
            if (phase + half) % 2 == 0:
                return p1
            return p2

        def rdma(phase, sub, half, sbuf):
            return pltpu.make_async_remote_copy(
                src_ref=sbuf.at[sub, half],
                dst_ref=rbuf.at[phase, sub, half],
                send_sem=send_sems.at[phase, sub, half],
                recv_sem=recv_sems.at[phase, sub, half],
                device_id=(partner(phase, half),),
                device_id_type=pl.DeviceIdType.MESH,
            )

        @pl.when(s == 0)
        def _():
            bsem = pltpu.get_barrier_semaphore()
            pl.semaphore_signal(
                bsem, inc=1, device_id=(p1,),
                device_id_type=pl.DeviceIdType.MESH,
            )
            pl.semaphore_signal(
                bsem, inc=1, device_id=(p2,),
                device_id_type=pl.DeviceIdType.MESH,
            )
            pl.semaphore_wait(bsem, 2)

        @pl.when((s >= 2) & (s <= nb))
        def _():
            pl.semaphore_wait(c1, 1)
            pl.semaphore_wait(c2, 1)

        @pl.when((s >= 1) & (s <= nb))
        def _():
            for sub in range(4):
                rdma(0, sub, 0, sbuf0).start()
                rdma(0, sub, 1, sbuf0).start()

        @pl.when(s >= 1)
        def _():
            for sub in range(4):
                for half in (0, 1):
                    rdma(0, sub, half, sbuf0).wait()
                for half in (0, 1):
                    sbuf1[sub, half] = (
                        sbuf0[sub, half].astype(F32)
                        + rbuf[0, sub, half].astype(F32)
                    ).astype(BF16)
                for half in (0, 1):
                    rdma(1, sub, half, sbuf1).start()

        @pl.when((s >= 1) & (s <= nb - 1))
        def _():
            pl.semaphore_signal(
                c1, inc=1, device_id=(p1,),
                device_id_type=pl.DeviceIdType.MESH,
            )
            pl.semaphore_signal(
                c2, inc=1, device_id=(p2,),
                device_id_type=pl.DeviceIdType.MESH,
            )

        @pl.when(s <= nb - 1)
        def _():
            for sub in range(4):
                r0 = sub * SUB
                sbuf0[sub, 0] = jnp.dot(
                    x_ref[r0:r0 + SUB, :], w_ref[:, :NHALF],
                    preferred_element_type=F32,
                ).astype(BF16)
                sbuf0[sub, 1] = jnp.dot(
                    x_ref[r0:r0 + SUB, :], w_ref[:, NHALF:],
                    preferred_element_type=F32,
                ).astype(BF16)

        @pl.when(s >= 1)
        def _():
            for sub in range(4):
                r0 = sub * SUB
                for half in (0, 1):
                    rdma(1, sub, half, sbuf1).wait()
                yA = (
                    sbuf1[sub, 0].astype(F32) + rbuf[1, sub, 0].astype(F32)
                )
                yB = (
                    sbuf1[sub, 1].astype(F32) + rbuf[1, sub, 1].astype(F32)
                )
                o_ref[r0:r0 + SUB, :NHALF] = jax.nn.gelu(
                    yA, approximate=True
                )
                o_ref[r0:r0 + SUB, NHALF:] = jax.nn.gelu(
                    yB, approximate=True
                )

    return pl.pallas_call(
        body,
        grid=(nb + 1,),
        out_shape=jax.ShapeDtypeStruct((m, n), F32),
        in_specs=[
            pl.BlockSpec((M_BLK, k), lambda s: (jnp.minimum(s, nb - 1), 0)),
            pl.BlockSpec((k2, n), lambda s: (0, 0)),
        ],
        out_specs=pl.BlockSpec(
            (M_BLK, n), lambda s: (jnp.maximum(s - 1, 0), 0)
        ),
        scratch_shapes=[
            pltpu.VMEM((4, 2, SUB, NHALF), BF16),
            pltpu.VMEM((4, 2, SUB, NHALF), BF16),
            pltpu.VMEM((2, 4, 2, SUB, NHALF), BF16),
            pltpu.SemaphoreType.DMA((2, 4, 2)),
            pltpu.SemaphoreType.DMA((2, 4, 2)),
            pltpu.SemaphoreType.REGULAR,
            pltpu.SemaphoreType.REGULAR,
        ],
        compiler_params=pltpu.CompilerParams(
            vmem_limit_bytes=100 * 1024 * 1024,
            collective_id=0,
        ),
    )(x_bf, w_bf)
